# baseline (speedup 1.0000x reference)
import jax
import jax.numpy as jnp
from jax import lax
from jax.experimental import pallas as pl
from jax.experimental.pallas import tpu as pltpu
from jax.experimental.pallas import tpu_sc as plsc

_B = 4096
_S = 200
_D = 64
_N = _B * _S
_NW = 32
_PER_W = _N // _NW
_CH = 200
_NCH = _PER_W // _CH
_G = 40
_NG = _CH // _G
_LANES = 16
_QV = _D // _LANES


def _emb_body(idx_hbm, table_hbm, pe_hbm, out_hbm,
              idx_v0, idx_v1, rows_v0, rows_v1, out_v0, out_v1, pe_v,
              si0, si1, sg0, sg1, so0, so1):
    wid = lax.axis_index("s") * 2 + lax.axis_index("c")
    base = wid * _PER_W
    cbase = wid * _NCH

    def fire_idx(chunk, idx_v, sem):
        pltpu.async_copy(idx_hbm.at[chunk], idx_v, sem)

    def wait_idx(idx_v, sem):
        pltpu.make_async_copy(idx_hbm.at[0], idx_v, sem).wait()

    def fire_gathers(idx_v, rows_v, sem):
        for j in range(_NG):
            pltpu.async_copy(
                table_hbm.at[idx_v.at[j]],
                rows_v.at[pl.ds(j * _G, _G)],
                sem,
            )

    def wait_gathers(rows_v, sem):
        pltpu.make_async_copy(table_hbm.at[pl.ds(0, _CH)], rows_v, sem).wait()

    def fire_out(row0, out_v, sem):
        pltpu.async_copy(out_v, out_hbm.at[pl.ds(row0, _CH)], sem)

    def wait_out(out_v, sem):
        pltpu.make_async_copy(out_v, out_hbm.at[pl.ds(0, _CH)], sem).wait()

    def add_pe(rows_v, out_v):
        @plsc.parallel_loop(0, _CH, unroll=4)
        def _(r):
            for q in range(_QV):
                sl = pl.ds(q * _LANES, _LANES)
                out_v[r, sl] = rows_v[r, sl] + pe_v[r, sl]

    pltpu.sync_copy(pe_hbm, pe_v)

    fire_idx(cbase, idx_v0, si0)
    fire_idx(cbase + 1, idx_v1, si1)
    wait_idx(idx_v0, si0)
    fire_gathers(idx_v0, rows_v0, sg0)

    def step(c, cur, nxt):
        idx_c, rows_c, out_c, si_c, sg_c, so_c = cur
        idx_n, rows_n, out_n, si_n, sg_n, so_n = nxt

        @pl.when(c + 1 < _NCH)
        def _():
            wait_idx(idx_n, si_n)
            fire_gathers(idx_n, rows_n, sg_n)

        wait_gathers(rows_c, sg_c)

        @pl.when(c + 2 < _NCH)
        def _():
            fire_idx(cbase + c + 2, idx_c, si_c)

        @pl.when(c > 1)
        def _():
            wait_out(out_c, so_c)

        add_pe(rows_c, out_c)
        fire_out(base + c * _CH, out_c, so_c)

    buf0 = (idx_v0, rows_v0, out_v0, si0, sg0, so0)
    buf1 = (idx_v1, rows_v1, out_v1, si1, sg1, so1)

    def pair(t, carry):
        step(2 * t, buf0, buf1)
        step(2 * t + 1, buf1, buf0)
        return carry

    lax.fori_loop(0, _NCH // 2, pair, 0)

    wait_out(out_v0, so0)
    wait_out(out_v1, so1)


@jax.jit
def _embed(idx3d, table, pe):
    mesh = plsc.VectorSubcoreMesh(core_axis_name="c", subcore_axis_name="s")
    f = pl.kernel(
        _emb_body,
        out_type=jax.ShapeDtypeStruct((_N, _D), jnp.float32),
        mesh=mesh,
        scratch_types=[
            pltpu.VMEM((_NG, _G), jnp.int32),
            pltpu.VMEM((_NG, _G), jnp.int32),
            pltpu.VMEM((_CH, _D), jnp.float32),
            pltpu.VMEM((_CH, _D), jnp.float32),
            pltpu.VMEM((_CH, _D), jnp.float32),
            pltpu.VMEM((_CH, _D), jnp.float32),
            pltpu.VMEM((_S, _D), jnp.float32),
            pltpu.SemaphoreType.DMA,
            pltpu.SemaphoreType.DMA,
            pltpu.SemaphoreType.DMA,
            pltpu.SemaphoreType.DMA,
            pltpu.SemaphoreType.DMA,
            pltpu.SemaphoreType.DMA,
        ],
        compiler_params=pltpu.CompilerParams(use_tc_tiling_on_sc=False),
    )
    return f(idx3d, table, pe)


def kernel(inputs, table, pos_encoding):
    idx3d = inputs.reshape(_N // _CH, _NG, _G).astype(jnp.int32)
    out = _embed(idx3d, table, pos_encoding[: _S])
    return out.reshape(_B, _S, _D)


# speedup vs baseline: 1.4687x; 1.4687x over previous
import jax
import jax.numpy as jnp
from jax import lax
from jax.experimental import pallas as pl
from jax.experimental.pallas import tpu as pltpu
from jax.experimental.pallas import tpu_sc as plsc

_B = 4096
_S = 200
_D = 64
_DP = 128
_N = _B * _S
_NW = 32
_PER_W = _N // _NW
_CH = 200
_CHP = _CH // 2
_NCH = _PER_W // _CH
_G = 40
_NG = _CH // _G
_LANES = 16
_QV = _D // _LANES


def _emb_body(idx_hbm, table_hbm, pe_hbm, out_hbm,
              idx_v0, idx_v1, rows_v0, rows_v1, out_v0, out_v1, pe_v,
              si0, si1, sg0, sg1, so0, so1):
    wid = lax.axis_index("s") * 2 + lax.axis_index("c")
    cbase = wid * _NCH

    def fire_idx(chunk, idx_v, sem):
        pltpu.async_copy(idx_hbm.at[chunk], idx_v, sem)

    def wait_idx(idx_v, sem):
        pltpu.make_async_copy(idx_hbm.at[0], idx_v, sem).wait()

    def fire_gathers(idx_v, rows_v, sem):
        for j in range(_NG):
            pltpu.async_copy(
                table_hbm.at[idx_v.at[j]],
                rows_v.at[pl.ds(j * _G, _G)],
                sem,
            )

    def wait_gathers(rows_v, sem):
        pltpu.make_async_copy(table_hbm.at[pl.ds(0, _CH)], rows_v, sem).wait()

    def fire_out(chunk, out_v, sem):
        pltpu.async_copy(out_v, out_hbm.at[chunk], sem)

    def wait_out(out_v, sem):
        pltpu.make_async_copy(out_v, out_hbm.at[0], sem).wait()

    def add_pe(rows_v, out_v):
        @plsc.parallel_loop(0, _CHP, unroll=4)
        def _(r):
            for h in range(2):
                for q in range(_QV):
                    dst = pl.ds(h * _D + q * _LANES, _LANES)
                    src = pl.ds(q * _LANES, _LANES)
                    out_v[r, dst] = rows_v[2 * r + h, src] + pe_v[r, dst]

    pltpu.sync_copy(pe_hbm, pe_v)

    fire_idx(cbase, idx_v0, si0)
    fire_idx(cbase + 1, idx_v1, si1)
    wait_idx(idx_v0, si0)
    fire_gathers(idx_v0, rows_v0, sg0)

    def step(c, cur, nxt):
        idx_c, rows_c, out_c, si_c, sg_c, so_c = cur
        idx_n, rows_n, out_n, si_n, sg_n, so_n = nxt

        @pl.when(c + 1 < _NCH)
        def _():
            wait_idx(idx_n, si_n)
            fire_gathers(idx_n, rows_n, sg_n)

        wait_gathers(rows_c, sg_c)

        @pl.when(c + 2 < _NCH)
        def _():
            fire_idx(cbase + c + 2, idx_c, si_c)

        @pl.when(c > 1)
        def _():
            wait_out(out_c, so_c)

        add_pe(rows_c, out_c)
        fire_out(cbase + c, out_c, so_c)

    buf0 = (idx_v0, rows_v0, out_v0, si0, sg0, so0)
    buf1 = (idx_v1, rows_v1, out_v1, si1, sg1, so1)

    def pair(t, carry):
        step(2 * t, buf0, buf1)
        step(2 * t + 1, buf1, buf0)
        return carry

    lax.fori_loop(0, _NCH // 2, pair, 0)

    wait_out(out_v0, so0)
    wait_out(out_v1, so1)


@jax.jit
def _embed(idx3d, table128, pe_packed):
    mesh = plsc.VectorSubcoreMesh(core_axis_name="c", subcore_axis_name="s")
    f = pl.kernel(
        _emb_body,
        out_type=jax.ShapeDtypeStruct((_N // _CH, _CHP, _DP), jnp.float32),
        mesh=mesh,
        scratch_types=[
            pltpu.VMEM((_NG, _G), jnp.int32),
            pltpu.VMEM((_NG, _G), jnp.int32),
            pltpu.VMEM((_CH, _DP), jnp.float32),
            pltpu.VMEM((_CH, _DP), jnp.float32),
            pltpu.VMEM((_CHP, _DP), jnp.float32),
            pltpu.VMEM((_CHP, _DP), jnp.float32),
            pltpu.VMEM((_CHP, _DP), jnp.float32),
            pltpu.SemaphoreType.DMA,
            pltpu.SemaphoreType.DMA,
            pltpu.SemaphoreType.DMA,
            pltpu.SemaphoreType.DMA,
            pltpu.SemaphoreType.DMA,
            pltpu.SemaphoreType.DMA,
        ],
        compiler_params=pltpu.CompilerParams(use_tc_tiling_on_sc=True),
    )
    return f(idx3d, table128, pe_packed)


def kernel(inputs, table, pos_encoding):
    idx3d = inputs.reshape(_N // _CH, _NG, _G).astype(jnp.int32)
    sel = jnp.eye(_D, _DP, dtype=jnp.float32)
    table128 = jax.lax.dot_general(
        table, sel, (((1,), (0,)), ((), ())),
        precision=jax.lax.Precision.HIGHEST,
    )
    pe_packed = pos_encoding[: _S].reshape(_CHP, _DP)
    out = _embed(idx3d, table128, pe_packed)
    return out.reshape(_B, _S, _D)
